# baseline (speedup 1.0000x reference)
import functools

import jax
import jax.numpy as jnp
from jax import lax
from jax.experimental import pallas as pl
from jax.experimental.pallas import tpu as pltpu
from jax.experimental.pallas import tpu_sc as plsc

_N = 16384
_D = 4096

_NSC = 4096
_NT = _N - _NSC
_TBLK = 1024
_TGRID = _NT // _TBLK

_L = 16
_NC = 2
_NS = 16
_NW = _NC * _NS
_RPT = _NSC // _NW
_RB = 8
_NBLK = _RPT // _RB
_G = 8
_NG = _D // (_G * _L)

_mesh = plsc.VectorSubcoreMesh(core_axis_name="c", subcore_axis_name="s")

_GDN = lax.GatherDimensionNumbers(
    offset_dims=(), collapsed_slice_dims=(0,), start_index_map=(0,))


def _bcast_lane(v, r):
    idx = jnp.full((_L, 1), r, jnp.int32)
    return lax.gather(v, idx, _GDN, slice_sizes=(1,),
                      mode=lax.GatherScatterMode.PROMISE_IN_BOUNDS)


@functools.partial(
    pl.kernel,
    mesh=_mesh,
    out_type=jax.ShapeDtypeStruct((_NW, 3 * _D), jnp.float32),
    scratch_types=[
        pltpu.VMEM((_RB, _D), jnp.float32),
        pltpu.VMEM((_RB, _D), jnp.float32),
        pltpu.VMEM((3 * _D,), jnp.float32),
        pltpu.VMEM((_RPT + _L,), jnp.float32),
        pltpu.VMEM((_RPT + _L,), jnp.float32),
        pltpu.SemaphoreType.DMA,
        pltpu.SemaphoreType.DMA,
    ],
)
def _sc_partial_sums(p_hbm, y_hbm, yp_hbm, out_hbm, buf0, buf1, acc, w1, w2,
                     sem0, sem1):
    wid = lax.axis_index("s") * _NC + lax.axis_index("c")
    base = _NT + wid * _RPT

    pltpu.sync_copy(y_hbm.at[pl.ds(base, _RPT)], w1.at[pl.ds(0, _RPT)])
    pltpu.sync_copy(yp_hbm.at[pl.ds(base, _RPT)], w2.at[pl.ds(0, _RPT)])

    zeros16 = jnp.zeros((_L,), jnp.float32)
    ones16 = jnp.ones((_L,), jnp.float32)

    def _wbody(i, _):
        o = i * _L
        w1[pl.ds(o, _L)] = jnp.where(w1[pl.ds(o, _L)] >= 0.0, ones16, zeros16)
        w2[pl.ds(o, _L)] = jnp.where(w2[pl.ds(o, _L)] >= 0.0, ones16, zeros16)
        return _
    lax.fori_loop(0, _RPT // _L, _wbody, None)
    w1[pl.ds(_RPT, _L)] = zeros16
    w2[pl.ds(_RPT, _L)] = zeros16

    def _zbody(i, _):
        acc[pl.ds(i * _L, _L)] = zeros16
        return _
    lax.fori_loop(0, 3 * _D // _L, _zbody, None)

    def _start(blk, buf, sem):
        pltpu.async_copy(p_hbm.at[pl.ds(base + blk * _RB, _RB)], buf, sem)

    def _wait(blk, buf, sem):
        pltpu.make_async_copy(
            p_hbm.at[pl.ds(base + blk * _RB, _RB)], buf, sem).wait()

    def _accum(buf, blk):
        w1v = w1[pl.ds(blk * _RB, _L)]
        w2v = w2[pl.ds(blk * _RB, _L)]

        def _g_body(g, _g):
            col0 = g * (_G * _L)
            a = ([acc[pl.ds(col0 + k * _L, _L)] for k in range(_G)]
                 + [acc[pl.ds(_D + col0 + k * _L, _L)] for k in range(_G)]
                 + [acc[pl.ds(2 * _D + col0 + k * _L, _L)] for k in range(_G)])
            for row in range(_RB):
                b1 = _bcast_lane(w1v, row)
                b2 = _bcast_lane(w2v, row)
                for k in range(_G):
                    v = buf[row, pl.ds(col0 + k * _L, _L)]
                    a[k] = a[k] + v
                    a[_G + k] = a[_G + k] + v * b1
                    a[2 * _G + k] = a[2 * _G + k] + v * b2
            for k in range(_G):
                acc[pl.ds(col0 + k * _L, _L)] = a[k]
                acc[pl.ds(_D + col0 + k * _L, _L)] = a[_G + k]
                acc[pl.ds(2 * _D + col0 + k * _L, _L)] = a[2 * _G + k]
            return _g
        lax.fori_loop(0, _NG, _g_body, None)

    _start(0, buf0, sem0)

    def _body(i, _):
        _start(2 * i + 1, buf1, sem1)
        _wait(2 * i, buf0, sem0)
        _accum(buf0, 2 * i)

        @pl.when(i < _NBLK // 2 - 1)
        def _():
            _start(2 * i + 2, buf0, sem0)

        _wait(2 * i + 1, buf1, sem1)
        _accum(buf1, 2 * i + 1)
        return _

    lax.fori_loop(0, _NBLK // 2, _body, None)

    pltpu.sync_copy(acc, out_hbm.at[wid])


def _tc_sums_body(p_ref, y_ref, yp_ref, out_ref, acc_ref):
    j = pl.program_id(0)

    @pl.when(j == 0)
    def _init():
        acc_ref[...] = jnp.zeros_like(acc_ref)

    blk = p_ref[...]
    y = y_ref[...]
    yp = yp_ref[...]
    w_pos = (y >= 0).astype(jnp.float32)
    w_pp = (yp >= 0).astype(jnp.float32)
    ones = jnp.ones_like(w_pos)
    W = jnp.stack([ones, w_pos, w_pp], axis=0)
    acc_ref[0:3, :] += jnp.dot(W, blk, preferred_element_type=jnp.float32)

    @pl.when(j == _TGRID - 1)
    def _write():
        out_ref[...] = acc_ref[0:3, :]


def _finish_body(tc_ref, sc_ref, y_ref, yp_ref, out_ref):
    red = jnp.sum(sc_ref[...], axis=0)
    s_all = tc_ref[0, :] + red[0:_D]
    s_pos = tc_ref[1, :] + red[_D:2 * _D]
    s_pp = tc_ref[2, :] + red[2 * _D:3 * _D]
    y = y_ref[...]
    yp = yp_ref[...]
    n = jnp.float32(_N)
    n_pos = jnp.sum((y >= 0.0).astype(jnp.float32))
    n_pp = jnp.sum((yp >= 0.0).astype(jnp.float32))
    n_neg = n - n_pos

    pos_avg = s_pos / n_pos
    neg_avg = (s_all - s_pos) / n_neg
    pos_avg_p = s_pp / n_pp
    neg_avg_p = (s_all - s_pp) / (n - n_pp)

    def one_minus_cos(a, b):
        dot = jnp.sum(a * b)
        na = jnp.sqrt(jnp.sum(a * a))
        nb = jnp.sqrt(jnp.sum(b * b))
        return 1.0 - dot / jnp.maximum(na * nb, 1e-8)

    cp = one_minus_cos(pos_avg, pos_avg_p)
    cn = one_minus_cos(neg_avg, neg_avg_p)
    out_ref[0] = n_pos * cp / n + n_neg * cn / n


@jax.jit
def kernel(p_v, y, y_pred):
    sc_partial = _sc_partial_sums(p_v, y, y_pred)
    tc_sums = pl.pallas_call(
        _tc_sums_body,
        grid=(_TGRID,),
        in_specs=[
            pl.BlockSpec((_TBLK, _D), lambda j: (j, 0)),
            pl.BlockSpec((_TBLK,), lambda j: (j,)),
            pl.BlockSpec((_TBLK,), lambda j: (j,)),
        ],
        out_specs=pl.BlockSpec((3, _D), lambda j: (0, 0)),
        out_shape=jax.ShapeDtypeStruct((3, _D), jnp.float32),
        scratch_shapes=[pltpu.VMEM((8, _D), jnp.float32)],
    )(p_v, y, y_pred)
    out = pl.pallas_call(
        _finish_body,
        out_specs=pl.BlockSpec(memory_space=pltpu.SMEM),
        out_shape=jax.ShapeDtypeStruct((1,), jnp.float32),
    )(tc_sums, sc_partial, y, y_pred)
    return out


# speedup vs baseline: 1.0087x; 1.0087x over previous
import functools

import jax
import jax.numpy as jnp
from jax import lax
from jax.experimental import pallas as pl
from jax.experimental.pallas import tpu as pltpu
from jax.experimental.pallas import tpu_sc as plsc

_N = 16384
_D = 4096

_NSC = 2048
_NT = _N - _NSC
_TBLK = 1024
_TGRID = _NT // _TBLK

_L = 16
_NC = 2
_NS = 16
_NW = _NC * _NS
_RPT = _NSC // _NW
_RB = 8
_NBLK = _RPT // _RB
_G = 8
_NG = _D // (_G * _L)

_mesh = plsc.VectorSubcoreMesh(core_axis_name="c", subcore_axis_name="s")

_GDN = lax.GatherDimensionNumbers(
    offset_dims=(), collapsed_slice_dims=(0,), start_index_map=(0,))


def _bcast_lane(v, r):
    idx = jnp.full((_L, 1), r, jnp.int32)
    return lax.gather(v, idx, _GDN, slice_sizes=(1,),
                      mode=lax.GatherScatterMode.PROMISE_IN_BOUNDS)


@functools.partial(
    pl.kernel,
    mesh=_mesh,
    out_type=jax.ShapeDtypeStruct((_NW, 3 * _D), jnp.float32),
    scratch_types=[
        pltpu.VMEM((_RB, _D), jnp.float32),
        pltpu.VMEM((_RB, _D), jnp.float32),
        pltpu.VMEM((3 * _D,), jnp.float32),
        pltpu.VMEM((_RPT + _L,), jnp.float32),
        pltpu.VMEM((_RPT + _L,), jnp.float32),
        pltpu.SemaphoreType.DMA,
        pltpu.SemaphoreType.DMA,
    ],
)
def _sc_partial_sums(p_hbm, y_hbm, yp_hbm, out_hbm, buf0, buf1, acc, w1, w2,
                     sem0, sem1):
    wid = lax.axis_index("s") * _NC + lax.axis_index("c")
    base = _NT + wid * _RPT

    pltpu.sync_copy(y_hbm.at[pl.ds(base, _RPT)], w1.at[pl.ds(0, _RPT)])
    pltpu.sync_copy(yp_hbm.at[pl.ds(base, _RPT)], w2.at[pl.ds(0, _RPT)])

    zeros16 = jnp.zeros((_L,), jnp.float32)
    ones16 = jnp.ones((_L,), jnp.float32)

    def _wbody(i, _):
        o = i * _L
        w1[pl.ds(o, _L)] = jnp.where(w1[pl.ds(o, _L)] >= 0.0, ones16, zeros16)
        w2[pl.ds(o, _L)] = jnp.where(w2[pl.ds(o, _L)] >= 0.0, ones16, zeros16)
        return _
    lax.fori_loop(0, _RPT // _L, _wbody, None)
    w1[pl.ds(_RPT, _L)] = zeros16
    w2[pl.ds(_RPT, _L)] = zeros16

    def _zbody(i, _):
        acc[pl.ds(i * _L, _L)] = zeros16
        return _
    lax.fori_loop(0, 3 * _D // _L, _zbody, None)

    def _start(blk, buf, sem):
        pltpu.async_copy(p_hbm.at[pl.ds(base + blk * _RB, _RB)], buf, sem)

    def _wait(blk, buf, sem):
        pltpu.make_async_copy(
            p_hbm.at[pl.ds(base + blk * _RB, _RB)], buf, sem).wait()

    def _accum(buf, blk):
        w1v = w1[pl.ds(blk * _RB, _L)]
        w2v = w2[pl.ds(blk * _RB, _L)]

        def _g_body(g, _g):
            col0 = g * (_G * _L)
            a = ([acc[pl.ds(col0 + k * _L, _L)] for k in range(_G)]
                 + [acc[pl.ds(_D + col0 + k * _L, _L)] for k in range(_G)]
                 + [acc[pl.ds(2 * _D + col0 + k * _L, _L)] for k in range(_G)])
            for row in range(_RB):
                b1 = _bcast_lane(w1v, row)
                b2 = _bcast_lane(w2v, row)
                for k in range(_G):
                    v = buf[row, pl.ds(col0 + k * _L, _L)]
                    a[k] = a[k] + v
                    a[_G + k] = a[_G + k] + v * b1
                    a[2 * _G + k] = a[2 * _G + k] + v * b2
            for k in range(_G):
                acc[pl.ds(col0 + k * _L, _L)] = a[k]
                acc[pl.ds(_D + col0 + k * _L, _L)] = a[_G + k]
                acc[pl.ds(2 * _D + col0 + k * _L, _L)] = a[2 * _G + k]
            return _g
        lax.fori_loop(0, _NG, _g_body, None)

    _start(0, buf0, sem0)

    def _body(i, _):
        _start(2 * i + 1, buf1, sem1)
        _wait(2 * i, buf0, sem0)
        _accum(buf0, 2 * i)

        @pl.when(i < _NBLK // 2 - 1)
        def _():
            _start(2 * i + 2, buf0, sem0)

        _wait(2 * i + 1, buf1, sem1)
        _accum(buf1, 2 * i + 1)
        return _

    lax.fori_loop(0, _NBLK // 2, _body, None)

    pltpu.sync_copy(acc, out_hbm.at[wid])


def _tc_sums_body(p_ref, y_ref, yp_ref, out_ref, acc_ref):
    j = pl.program_id(0)

    @pl.when(j == 0)
    def _init():
        acc_ref[...] = jnp.zeros_like(acc_ref)

    blk = p_ref[...]
    y = y_ref[...]
    yp = yp_ref[...]
    w_pos = (y >= 0).astype(jnp.float32)
    w_pp = (yp >= 0).astype(jnp.float32)
    ones = jnp.ones_like(w_pos)
    W = jnp.stack([ones, w_pos, w_pp], axis=0)
    acc_ref[0:3, :] += jnp.dot(W, blk, preferred_element_type=jnp.float32)

    @pl.when(j == _TGRID - 1)
    def _write():
        out_ref[...] = acc_ref[0:3, :]


def _finish_body(tc_ref, sc_ref, y_ref, yp_ref, out_ref):
    red = jnp.sum(sc_ref[...], axis=0)
    s_all = tc_ref[0, :] + red[0:_D]
    s_pos = tc_ref[1, :] + red[_D:2 * _D]
    s_pp = tc_ref[2, :] + red[2 * _D:3 * _D]
    y = y_ref[...]
    yp = yp_ref[...]
    n = jnp.float32(_N)
    n_pos = jnp.sum((y >= 0.0).astype(jnp.float32))
    n_pp = jnp.sum((yp >= 0.0).astype(jnp.float32))
    n_neg = n - n_pos

    pos_avg = s_pos / n_pos
    neg_avg = (s_all - s_pos) / n_neg
    pos_avg_p = s_pp / n_pp
    neg_avg_p = (s_all - s_pp) / (n - n_pp)

    def one_minus_cos(a, b):
        dot = jnp.sum(a * b)
        na = jnp.sqrt(jnp.sum(a * a))
        nb = jnp.sqrt(jnp.sum(b * b))
        return 1.0 - dot / jnp.maximum(na * nb, 1e-8)

    cp = one_minus_cos(pos_avg, pos_avg_p)
    cn = one_minus_cos(neg_avg, neg_avg_p)
    out_ref[0] = n_pos * cp / n + n_neg * cn / n


@jax.jit
def kernel(p_v, y, y_pred):
    sc_partial = _sc_partial_sums(p_v, y, y_pred)
    tc_sums = pl.pallas_call(
        _tc_sums_body,
        grid=(_TGRID,),
        in_specs=[
            pl.BlockSpec((_TBLK, _D), lambda j: (j, 0)),
            pl.BlockSpec((_TBLK,), lambda j: (j,)),
            pl.BlockSpec((_TBLK,), lambda j: (j,)),
        ],
        out_specs=pl.BlockSpec((3, _D), lambda j: (0, 0)),
        out_shape=jax.ShapeDtypeStruct((3, _D), jnp.float32),
        scratch_shapes=[pltpu.VMEM((8, _D), jnp.float32)],
    )(p_v, y, y_pred)
    out = pl.pallas_call(
        _finish_body,
        out_specs=pl.BlockSpec(memory_space=pltpu.SMEM),
        out_shape=jax.ShapeDtypeStruct((1,), jnp.float32),
    )(tc_sums, sc_partial, y, y_pred)
    return out
